# baseline (speedup 1.0000x reference)
import functools

import jax
import jax.numpy as jnp
from jax import lax
from jax.experimental import pallas as pl
from jax.experimental.pallas import tpu as pltpu
from jax.experimental.pallas import tpu_sc as plsc

D = 64
PK = 128
L = 16
QL = 128
HALF = 250000


@functools.lru_cache(maxsize=None)
def _make_sc_kernel(B):
    info = plsc.get_sparse_core_info()
    NC, NS = info.num_cores, info.num_subcores
    NW = NC * NS
    bpw = B // NW
    nq = bpw // QL
    assert B % (NW * QL) == 0

    mesh = plsc.VectorSubcoreMesh(core_axis_name="c", subcore_axis_name="s")

    @functools.partial(
        pl.kernel,
        mesh=mesh,
        compiler_params=pltpu.CompilerParams(
            needs_layout_passes=False, use_tc_tiling_on_sc=False),
        out_type=jax.ShapeDtypeStruct((B,), jnp.float32),
        scratch_types=[
            pltpu.VMEM((bpw,), jnp.int32),
            pltpu.VMEM((bpw,), jnp.int32),
            pltpu.VMEM((1, QL), jnp.int32),
            pltpu.VMEM((1, QL), jnp.int32),
            pltpu.VMEM((1, QL), jnp.int32),
            pltpu.VMEM((1, QL), jnp.int32),
            pltpu.VMEM((QL, PK), jnp.float32),
            pltpu.VMEM((QL, PK), jnp.float32),
            pltpu.VMEM((QL, PK), jnp.float32),
            pltpu.VMEM((QL, PK), jnp.float32),
            pltpu.VMEM((bpw,), jnp.float32),
            pltpu.VMEM((L,), jnp.float32),
            pltpu.VMEM((L,), jnp.float32),
            pltpu.SemaphoreType.DMA,
        ],
    )
    def sc_kernel(idx_t_hbm, idx_c_hbm, tblA_hbm, tblB_hbm, w_hbm, b_hbm,
                  out_hbm, idx_t_v, idx_c_v, pta_v, ptb_v, pca_v, pcb_v,
                  rta_v, rtb_v, rca_v, rcb_v, out_v, w_v, b_v, sem):
        wid = lax.axis_index("s") * NC + lax.axis_index("c")
        base = wid * bpw

        pltpu.sync_copy(idx_t_hbm.at[wid], idx_t_v)
        pltpu.sync_copy(idx_c_hbm.at[wid], idx_c_v)
        pltpu.sync_copy(w_hbm, w_v)
        pltpu.sync_copy(b_hbm, b_v)

        wv = w_v[...]
        bv = b_v[...]
        lane_iota = lax.iota(jnp.int32, L)

        for q in range(nq):
            qoff = q * QL

            def pk_body(v, carry):
                s = pl.ds(v * L, L)
                rt = idx_t_v[pl.ds(qoff + v * L, L)] >> 1
                rc = idx_c_v[pl.ds(qoff + v * L, L)] >> 1
                pta_v[0, s] = jnp.minimum(rt, HALF - 1)
                ptb_v[0, s] = jnp.maximum(rt - HALF, 0)
                pca_v[0, s] = jnp.minimum(rc, HALF - 1)
                pcb_v[0, s] = jnp.maximum(rc - HALF, 0)
                return carry

            lax.fori_loop(0, QL // L, pk_body, 0)

            cps = [
                pltpu.async_copy(tblA_hbm.at[pta_v.at[0]], rta_v, sem),
                pltpu.async_copy(tblB_hbm.at[ptb_v.at[0]], rtb_v, sem),
                pltpu.async_copy(tblA_hbm.at[pca_v.at[0]], rca_v, sem),
                pltpu.async_copy(tblB_hbm.at[pcb_v.at[0]], rcb_v, sem),
            ]
            for cp in cps:
                cp.wait()

            def group_body(v, carry):
                s = pl.ds(qoff + v * L, L)
                it = idx_t_v[s]
                ic = idx_c_v[s]
                off_t = (it & 1) << 6
                off_c = (ic & 1) << 6
                mt = (it >> 1) < HALF
                mc = (ic >> 1) < HALF
                rows = jnp.full((L,), v * L, jnp.int32) + lane_iota

                def dim_body(d, acc):
                    ct = off_t + d
                    cc = off_c + d
                    vta = plsc.load_gather(rta_v, [rows, ct])
                    vtb = plsc.load_gather(rtb_v, [rows, ct])
                    vca = plsc.load_gather(rca_v, [rows, cc])
                    vcb = plsc.load_gather(rcb_v, [rows, cc])
                    vt = jnp.where(mt, vta, vtb)
                    vc = jnp.where(mc, vca, vcb)
                    return acc + vt * vc

                acc = lax.fori_loop(0, D, dim_body,
                                    jnp.zeros((L,), jnp.float32))
                z = acc * wv + bv
                out_v[pl.ds(qoff + v * L, L)] = 1.0 / (1.0 + jnp.exp(-z))
                return carry

            lax.fori_loop(0, QL // L, group_body, 0)

        pltpu.sync_copy(out_v, out_hbm.at[pl.ds(base, bpw)])

    return sc_kernel, NW


def kernel(input_target, input_context, embedding_table, dense_w, dense_b):
    B = input_target.shape[0]
    sc_kernel, NW = _make_sc_kernel(B)
    table_pk = embedding_table.reshape(2 * HALF, PK)
    tblA = table_pk[:HALF]
    tblB = table_pk[HALF:]
    idx_t = input_target.reshape(NW, B // NW).astype(jnp.int32)
    idx_c = input_context.reshape(NW, B // NW).astype(jnp.int32)
    w_arr = jnp.full((L,), dense_w[0, 0], jnp.float32)
    b_arr = jnp.full((L,), dense_b[0], jnp.float32)
    out = sc_kernel(idx_t, idx_c, tblA, tblB, w_arr, b_arr)
    return out.reshape(B, 1)


# speedup vs baseline: 2.4592x; 2.4592x over previous
import functools

import jax
import jax.numpy as jnp
from jax import lax
from jax.experimental import pallas as pl
from jax.experimental.pallas import tpu as pltpu
from jax.experimental.pallas import tpu_sc as plsc

D = 64
PK = 128
L = 16
QL = 128
NPK = 500000


@functools.lru_cache(maxsize=None)
def _make_sc_kernel(B):
    info = plsc.get_sparse_core_info()
    NC, NS = info.num_cores, info.num_subcores
    NW = NC * NS
    bpw = B // NW
    nq = bpw // QL
    assert B % (NW * QL) == 0

    mesh = plsc.VectorSubcoreMesh(core_axis_name="c", subcore_axis_name="s")

    @functools.partial(
        pl.kernel,
        mesh=mesh,
        compiler_params=pltpu.CompilerParams(
            needs_layout_passes=False, use_tc_tiling_on_sc=False),
        out_type=jax.ShapeDtypeStruct((B,), jnp.float32),
        scratch_types=[
            pltpu.VMEM((bpw,), jnp.int32),
            pltpu.VMEM((bpw,), jnp.int32),
            pltpu.VMEM((1, QL), jnp.int32),
            pltpu.VMEM((1, QL), jnp.int32),
            pltpu.VMEM((QL, PK), jnp.float32),
            pltpu.VMEM((QL, PK), jnp.float32),
            pltpu.VMEM((bpw,), jnp.float32),
            pltpu.VMEM((L,), jnp.float32),
            pltpu.VMEM((L,), jnp.float32),
            pltpu.SemaphoreType.DMA,
        ],
    )
    def sc_kernel(idx_t_hbm, idx_c_hbm, tbl_hbm, w_hbm, b_hbm,
                  out_hbm, idx_t_v, idx_c_v, pt_v, pc_v,
                  rt_v, rc_v, out_v, w_v, b_v, sem):
        wid = lax.axis_index("s") * NC + lax.axis_index("c")
        base = wid * bpw

        pltpu.sync_copy(idx_t_hbm.at[wid], idx_t_v)
        pltpu.sync_copy(idx_c_hbm.at[wid], idx_c_v)
        pltpu.sync_copy(w_hbm, w_v)
        pltpu.sync_copy(b_hbm, b_v)

        wv = w_v[...]
        bv = b_v[...]
        lane_iota = lax.iota(jnp.int32, L)

        for q in range(nq):
            qoff = q * QL

            def pk_body(v, carry):
                s = pl.ds(v * L, L)
                pt_v[0, s] = idx_t_v[pl.ds(qoff + v * L, L)] >> 1
                pc_v[0, s] = idx_c_v[pl.ds(qoff + v * L, L)] >> 1
                return carry

            lax.fori_loop(0, QL // L, pk_body, 0)

            cps = [
                pltpu.async_copy(tbl_hbm.at[pt_v.at[0]], rt_v, sem),
                pltpu.async_copy(tbl_hbm.at[pc_v.at[0]], rc_v, sem),
            ]
            for cp in cps:
                cp.wait()

            def group_body(v, carry):
                s = pl.ds(qoff + v * L, L)
                off_t = (idx_t_v[s] & 1) << 6
                off_c = (idx_c_v[s] & 1) << 6
                rows = jnp.full((L,), v * L, jnp.int32) + lane_iota

                def dim_body(d, acc):
                    vt = plsc.load_gather(rt_v, [rows, off_t + d])
                    vc = plsc.load_gather(rc_v, [rows, off_c + d])
                    return acc + vt * vc

                acc = lax.fori_loop(0, D, dim_body,
                                    jnp.zeros((L,), jnp.float32))
                z = acc * wv + bv
                out_v[pl.ds(qoff + v * L, L)] = 1.0 / (1.0 + jnp.exp(-z))
                return carry

            lax.fori_loop(0, QL // L, group_body, 0)

        pltpu.sync_copy(out_v, out_hbm.at[pl.ds(base, bpw)])

    return sc_kernel, NW


def kernel(input_target, input_context, embedding_table, dense_w, dense_b):
    B = input_target.shape[0]
    sc_kernel, NW = _make_sc_kernel(B)
    table_pk = embedding_table.reshape(NPK, PK)
    idx_t = input_target.reshape(NW, B // NW).astype(jnp.int32)
    idx_c = input_context.reshape(NW, B // NW).astype(jnp.int32)
    w_arr = jnp.full((L,), dense_w[0, 0], jnp.float32)
    b_arr = jnp.full((L,), dense_b[0], jnp.float32)
    out = sc_kernel(idx_t, idx_c, table_pk, w_arr, b_arr)
    return out.reshape(B, 1)


# speedup vs baseline: 2.4618x; 1.0010x over previous
import functools

import jax
import jax.numpy as jnp
from jax import lax
from jax.experimental import pallas as pl
from jax.experimental.pallas import tpu as pltpu
from jax.experimental.pallas import tpu_sc as plsc

D = 64
PK = 128
L = 16
QL = 128
NPK = 500000


@functools.lru_cache(maxsize=None)
def _make_sc_kernel(B):
    info = plsc.get_sparse_core_info()
    NC, NS = info.num_cores, info.num_subcores
    NW = NC * NS
    bpw = B // NW
    nq = bpw // QL
    assert B % (NW * QL) == 0

    mesh = plsc.VectorSubcoreMesh(core_axis_name="c", subcore_axis_name="s")

    @functools.partial(
        pl.kernel,
        mesh=mesh,
        compiler_params=pltpu.CompilerParams(
            needs_layout_passes=False, use_tc_tiling_on_sc=False),
        out_type=jax.ShapeDtypeStruct((B,), jnp.float32),
        scratch_types=[
            pltpu.VMEM((bpw,), jnp.int32),
            pltpu.VMEM((bpw,), jnp.int32),
            pltpu.VMEM((1, QL), jnp.int32),
            pltpu.VMEM((1, QL), jnp.int32),
            pltpu.VMEM((QL, PK), jnp.float32),
            pltpu.VMEM((QL, PK), jnp.float32),
            pltpu.VMEM((bpw,), jnp.float32),
            pltpu.VMEM((L,), jnp.float32),
            pltpu.VMEM((L,), jnp.float32),
            pltpu.SemaphoreType.DMA,
        ],
    )
    def sc_kernel(idx_t_hbm, idx_c_hbm, tbl_hbm, w_hbm, b_hbm,
                  out_hbm, idx_t_v, idx_c_v, pt_v, pc_v,
                  rt_v, rc_v, out_v, w_v, b_v, sem):
        wid = lax.axis_index("s") * NC + lax.axis_index("c")
        base = wid * bpw

        pltpu.sync_copy(idx_t_hbm.at[wid], idx_t_v)
        pltpu.sync_copy(idx_c_hbm.at[wid], idx_c_v)
        pltpu.sync_copy(w_hbm, w_v)
        pltpu.sync_copy(b_hbm, b_v)

        wv = w_v[...]
        bv = b_v[...]
        lane_iota = lax.iota(jnp.int32, L)

        for q in range(nq):
            qoff = q * QL

            def pk_body(v, carry):
                s = pl.ds(v * L, L)
                pt_v[0, s] = idx_t_v[pl.ds(qoff + v * L, L)] >> 1
                pc_v[0, s] = idx_c_v[pl.ds(qoff + v * L, L)] >> 1
                return carry

            lax.fori_loop(0, QL // L, pk_body, 0)

            cps = [
                pltpu.async_copy(tbl_hbm.at[pt_v.at[0]], rt_v, sem),
                pltpu.async_copy(tbl_hbm.at[pc_v.at[0]], rc_v, sem),
            ]
            for cp in cps:
                cp.wait()

            def group_body(v, carry):
                s = pl.ds(qoff + v * L, L)
                off_t = (idx_t_v[s] & 1) << 6
                off_c = (idx_c_v[s] & 1) << 6
                rows = jnp.full((L,), v * L, jnp.int32) + lane_iota

                acc = jnp.zeros((L,), jnp.float32)
                for d in range(D):
                    vt = plsc.load_gather(rt_v, [rows, off_t + d])
                    vc = plsc.load_gather(rc_v, [rows, off_c + d])
                    acc = acc + vt * vc
                z = acc * wv + bv
                out_v[pl.ds(qoff + v * L, L)] = 1.0 / (1.0 + jnp.exp(-z))
                return carry

            lax.fori_loop(0, QL // L, group_body, 0)

        pltpu.sync_copy(out_v, out_hbm.at[pl.ds(base, bpw)])

    return sc_kernel, NW


def kernel(input_target, input_context, embedding_table, dense_w, dense_b):
    B = input_target.shape[0]
    sc_kernel, NW = _make_sc_kernel(B)
    table_pk = embedding_table.reshape(NPK, PK)
    idx_t = input_target.reshape(NW, B // NW).astype(jnp.int32)
    idx_c = input_context.reshape(NW, B // NW).astype(jnp.int32)
    w_arr = jnp.full((L,), dense_w[0, 0], jnp.float32)
    b_arr = jnp.full((L,), dense_b[0], jnp.float32)
    out = sc_kernel(idx_t, idx_c, table_pk, w_arr, b_arr)
    return out.reshape(B, 1)
